# baseline (speedup 1.0000x reference)
import functools

import jax
import jax.numpy as jnp
from jax import lax
from jax.experimental import pallas as pl
from jax.experimental.pallas import tpu as pltpu
from jax.experimental.pallas import tpu_sc as plsc

_LANES = 16
_IDX_CHUNK = 128


def _sc_hash_gather(n_tokens, seq, vocab, row_elems, n_workers, b_per_w):
    mod = vocab - 1
    n_idx_chunks = b_per_w // _IDX_CHUNK
    mesh = plsc.VectorSubcoreMesh(core_axis_name="c", subcore_axis_name="s")
    nc = 2

    @functools.partial(
        pl.kernel,
        mesh=mesh,
        out_type=jax.ShapeDtypeStruct((n_tokens, row_elems), jnp.float32),
        compiler_params=pltpu.CompilerParams(use_tc_tiling_on_sc=False),
        scratch_types=[
            pltpu.VMEM((b_per_w,), jnp.int32),
            pltpu.VMEM((b_per_w,), jnp.int32),
            pltpu.VMEM((n_idx_chunks, _IDX_CHUNK), jnp.int32),
            pltpu.VMEM((b_per_w, row_elems), jnp.float32),
            pltpu.SemaphoreType.DMA,
        ],
    )
    def sc_kernel(cur_hbm, prev_hbm, table_hbm, out_hbm,
                  cur_v, prev_v, idx_v, rows_v, sem):
        wid = lax.axis_index("s") * nc + lax.axis_index("c")
        base = wid * b_per_w
        pltpu.sync_copy(cur_hbm.at[pl.ds(base, b_per_w)], cur_v)
        pltpu.sync_copy(prev_hbm.at[pl.ds(base, b_per_w)], prev_v)

        lane = lax.iota(jnp.int32, _LANES)
        for i in range(b_per_w // _LANES):
            c = cur_v[pl.ds(i * _LANES, _LANES)]
            p = prev_v[pl.ds(i * _LANES, _LANES)]
            h = ((c * 36313) ^ (p * 27191)) % mod
            pos = base + i * _LANES + lane
            is_first = (pos & (seq - 1)) == 0
            idx = jnp.where(is_first, mod, h)
            idx_v[i // (_IDX_CHUNK // _LANES),
                  pl.ds((i % (_IDX_CHUNK // _LANES)) * _LANES, _LANES)] = idx

        copies = []
        for j in range(n_idx_chunks):
            copies.append(pltpu.async_copy(
                table_hbm.at[idx_v.at[jnp.int32(j)]],
                rows_v.at[pl.ds(jnp.int32(j * _IDX_CHUNK), _IDX_CHUNK)],
                sem))
        for cp in copies:
            cp.wait()
        pltpu.sync_copy(rows_v, out_hbm.at[pl.ds(base, b_per_w)])

    return sc_kernel


def _tc_proj(rows_ref, w_ref, scale_ref, out_ref):
    w = w_ref[...] * scale_ref[0, 0]
    out_ref[...] = lax.dot_general(
        rows_ref[...], w, (((1,), (1,)), ((), ())),
        preferred_element_type=jnp.float32)


def kernel(token_ids, table, W_proj, scale):
    batch, seq = token_ids.shape
    vocab, dim = table.shape
    model_dim = W_proj.shape[0]
    n = batch * seq

    tok = token_ids.astype(jnp.int32)
    cur = tok.reshape(n)
    prev = jnp.roll(tok, 1, axis=1).reshape(n)

    n_workers = 32
    b_per_w = n // n_workers
    rows = _sc_hash_gather(n, seq, vocab, dim, n_workers, b_per_w)(
        cur, prev, table)

    blk = 512
    out = pl.pallas_call(
        _tc_proj,
        grid=(n // blk, 1),
        in_specs=[
            pl.BlockSpec((blk, dim), lambda i, j: (i, j)),
            pl.BlockSpec((model_dim, dim), lambda i, j: (j, j)),
            pl.BlockSpec((1, 1), lambda i, j: (j, j),
                         memory_space=pltpu.SMEM),
        ],
        out_specs=pl.BlockSpec((blk, model_dim), lambda i, j: (i, j)),
        out_shape=jax.ShapeDtypeStruct((n, model_dim), jnp.float32),
    )(rows, W_proj, scale.reshape(1, 1))

    return out.reshape(batch, seq, model_dim)


# speedup vs baseline: 1.6280x; 1.6280x over previous
import functools

import jax
import jax.numpy as jnp
from jax import lax
from jax.experimental import pallas as pl
from jax.experimental.pallas import tpu as pltpu
from jax.experimental.pallas import tpu_sc as plsc

_LANES = 16
_IDX_CHUNK = 128


def _sc_hash_gather(n_tokens, seq, vocab, dim, n_workers, b_per_w):
    mod = vocab - 1
    mesh = plsc.VectorSubcoreMesh(core_axis_name="c", subcore_axis_name="s")
    nc = 2

    @functools.partial(
        pl.kernel,
        mesh=mesh,
        out_type=jax.ShapeDtypeStruct((n_tokens, dim), jnp.float32),
        compiler_params=pltpu.CompilerParams(needs_layout_passes=False),
        scratch_types=[
            pltpu.VMEM((b_per_w,), jnp.int32),
            pltpu.VMEM((b_per_w,), jnp.int32),
            pltpu.VMEM((b_per_w,), jnp.int32),
            pltpu.VMEM((b_per_w, dim), jnp.float32),
            pltpu.SemaphoreType.DMA,
        ],
    )
    def sc_kernel(cur_hbm, prev_hbm, table_hbm, out_hbm,
                  cur_v, prev_v, idx_v, rows_v, sem):
        wid = lax.axis_index("s") * nc + lax.axis_index("c")
        base = wid * b_per_w
        pltpu.sync_copy(cur_hbm.at[pl.ds(base, b_per_w)], cur_v)
        pltpu.sync_copy(prev_hbm.at[pl.ds(base, b_per_w)], prev_v)

        lane = lax.iota(jnp.int32, _LANES)
        for i in range(b_per_w // _LANES):
            c = cur_v[pl.ds(i * _LANES, _LANES)]
            p = prev_v[pl.ds(i * _LANES, _LANES)]
            h = ((c * 36313) ^ (p * 27191)) % mod
            pos = base + i * _LANES + lane
            is_first = (pos & (seq - 1)) == 0
            idx_v[pl.ds(i * _LANES, _LANES)] = jnp.where(is_first, mod, h)

        def issue(i, carry):
            rvec = plsc.load_gather(idx_v, [jnp.full((_LANES,), i, jnp.int32)])
            r = rvec[0]
            pltpu.make_async_copy(
                table_hbm.at[pl.ds(r, 1)],
                rows_v.at[pl.ds(i, 1)],
                sem).start()
            return carry

        lax.fori_loop(jnp.int32(0), jnp.int32(b_per_w), issue, jnp.int32(0))
        pltpu.make_async_copy(
            table_hbm.at[pl.ds(jnp.int32(0), b_per_w)], rows_v, sem).wait()
        pltpu.sync_copy(rows_v, out_hbm.at[pl.ds(base, b_per_w)])

    return sc_kernel


def _tc_proj(rows_ref, w_ref, scale_ref, out_ref):
    w = w_ref[...] * scale_ref[0, 0]
    out_ref[...] = lax.dot_general(
        rows_ref[...], w, (((1,), (1,)), ((), ())),
        preferred_element_type=jnp.float32)


def kernel(token_ids, table, W_proj, scale):
    batch, seq = token_ids.shape
    vocab, dim = table.shape
    model_dim = W_proj.shape[0]
    n = batch * seq

    tok = token_ids.astype(jnp.int32)
    cur = tok.reshape(n)
    prev = jnp.roll(tok, 1, axis=1).reshape(n)

    n_workers = 32
    b_per_w = n // n_workers
    rows = _sc_hash_gather(n, seq, vocab, dim, n_workers, b_per_w)(
        cur, prev, table)

    blk = 512
    out = pl.pallas_call(
        _tc_proj,
        grid=(n // blk, 1),
        in_specs=[
            pl.BlockSpec((blk, dim), lambda i, j: (i, j)),
            pl.BlockSpec((model_dim, dim), lambda i, j: (j, j)),
            pl.BlockSpec((1, 1), lambda i, j: (j, j),
                         memory_space=pltpu.SMEM),
        ],
        out_specs=pl.BlockSpec((blk, model_dim), lambda i, j: (i, j)),
        out_shape=jax.ShapeDtypeStruct((n, model_dim), jnp.float32),
    )(rows, W_proj, scale.reshape(1, 1))

    return out.reshape(batch, seq, model_dim)


# speedup vs baseline: 1.6500x; 1.0135x over previous
import functools

import jax
import jax.numpy as jnp
from jax import lax
from jax.experimental import pallas as pl
from jax.experimental.pallas import tpu as pltpu
from jax.experimental.pallas import tpu_sc as plsc

_LANES = 16


def _sc_hash_gather(n_tokens, seq, vocab, dim, n_workers, b_per_w):
    mod = vocab - 1
    mesh = plsc.VectorSubcoreMesh(core_axis_name="c", subcore_axis_name="s")
    nc = 2

    @functools.partial(
        pl.kernel,
        mesh=mesh,
        out_type=jax.ShapeDtypeStruct((n_tokens, dim), jnp.float32),
        scratch_types=[
            pltpu.VMEM((b_per_w,), jnp.int32),
            pltpu.VMEM((b_per_w,), jnp.int32),
            pltpu.VMEM((b_per_w,), jnp.int32),
            pltpu.SMEM((b_per_w,), jnp.int32),
            pltpu.VMEM_SHARED((16, b_per_w), jnp.int32),
            pltpu.VMEM((b_per_w, dim), jnp.float32),
            pltpu.SemaphoreType.DMA,
        ],
    )
    def sc_kernel(cur_hbm, prev_hbm, table_hbm, out_hbm,
                  cur_v, prev_v, idx_v, idx_s, idx_sh, rows_v, sem):
        wid = lax.axis_index("s") * nc + lax.axis_index("c")
        base = wid * b_per_w
        pltpu.sync_copy(cur_hbm.at[pl.ds(base, b_per_w)], cur_v)
        pltpu.sync_copy(prev_hbm.at[pl.ds(base, b_per_w)], prev_v)

        lane = lax.iota(jnp.int32, _LANES)
        for i in range(b_per_w // _LANES):
            c = cur_v[pl.ds(i * _LANES, _LANES)]
            p = prev_v[pl.ds(i * _LANES, _LANES)]
            h = ((c * 36313) ^ (p * 27191)) % mod
            pos = base + i * _LANES + lane
            is_first = (pos & (seq - 1)) == 0
            idx_v[pl.ds(i * _LANES, _LANES)] = jnp.where(is_first, mod, h)

        sid = lax.axis_index("s")
        pltpu.sync_copy(idx_v, idx_sh.at[sid])
        pltpu.sync_copy(idx_sh.at[sid], idx_s)

        def issue(i, carry):
            r = idx_s[i]
            pltpu.make_async_copy(
                table_hbm.at[pl.ds(r, 1)],
                rows_v.at[pl.ds(i, 1)],
                sem).start()
            return carry

        lax.fori_loop(jnp.int32(0), jnp.int32(b_per_w), issue, jnp.int32(0))
        pltpu.make_async_copy(
            table_hbm.at[pl.ds(jnp.int32(0), b_per_w)], rows_v, sem).wait()
        pltpu.sync_copy(rows_v, out_hbm.at[pl.ds(base, b_per_w)])

    return sc_kernel


def _tc_proj(rows_ref, w_ref, scale_ref, out_ref):
    w = w_ref[...] * scale_ref[0, 0]
    out_ref[...] = lax.dot_general(
        rows_ref[...], w, (((1,), (1,)), ((), ())),
        preferred_element_type=jnp.float32)


def kernel(token_ids, table, W_proj, scale):
    batch, seq = token_ids.shape
    vocab, dim = table.shape
    model_dim = W_proj.shape[0]
    n = batch * seq

    tok = token_ids.astype(jnp.int32)
    cur = tok.reshape(n)
    prev = jnp.roll(tok, 1, axis=1).reshape(n)

    n_workers = 32
    b_per_w = n // n_workers
    rows = _sc_hash_gather(n, seq, vocab, dim, n_workers, b_per_w)(
        cur, prev, table)

    blk = 512
    out = pl.pallas_call(
        _tc_proj,
        grid=(n // blk, 1),
        in_specs=[
            pl.BlockSpec((blk, dim), lambda i, j: (i, j)),
            pl.BlockSpec((model_dim, dim), lambda i, j: (j, j)),
            pl.BlockSpec((1, 1), lambda i, j: (j, j),
                         memory_space=pltpu.SMEM),
        ],
        out_specs=pl.BlockSpec((blk, model_dim), lambda i, j: (i, j)),
        out_shape=jax.ShapeDtypeStruct((n, model_dim), jnp.float32),
    )(rows, W_proj, scale.reshape(1, 1))

    return out.reshape(batch, seq, model_dim)
